# baseline (speedup 1.0000x reference)
import jax
import jax.numpy as jnp
from jax import lax
from jax.experimental import pallas as pl
from jax.experimental.pallas import tpu as pltpu, tpu_sc as plsc

N = 10000
E = 320000
D = 128
NPAD = 10240
NC = 2
NS = 16
NW = NC * NS
EPW = E // NW
CB = 8
NCHUNK = EPW // CB
NBLK = 25
BCH = NCHUNK // NBLK
DEPTH = 10
RPT = NPAD // NS


def _sc_body(x_hbm, src_hbm, dst_hbm, acc_out, cnt_out, *refs):
    idx_s, idx_d = refs[0], refs[1]
    sbufs = refs[2:2 + DEPTH]
    dbufs = refs[2 + DEPTH:2 + 2 * DEPTH]
    m0, m1, czero, ones_v = refs[2 + 2 * DEPTH:6 + 2 * DEPTH]
    sems = refs[6 + 2 * DEPTH:]
    ssems = sems[0:DEPTH]
    dsems = sems[DEPTH:2 * DEPTH]
    sem_m0, sem_m1, sem_c0, sem_c1 = sems[2 * DEPTH:2 * DEPTH + 4]
    acc_sh, cnt_sh = sems[2 * DEPTH + 4:]
    mbufs = (m0, m1)
    msems = (sem_m0, sem_m1)
    csems = (sem_c0, sem_c1)

    c = lax.axis_index("c")
    s = lax.axis_index("s")
    wid = c * NS + s

    def _zrow(r, _):
        for j in range(D // 16):
            m0[r, pl.ds(j * 16, 16)] = jnp.zeros((16,), jnp.float32)
        return 0
    lax.fori_loop(0, CB, _zrow, 0)

    def _zc(r, _):
        czero[pl.ds(r * 16, 16)] = jnp.zeros((16,), jnp.float32)
        return 0
    lax.fori_loop(0, RPT // 16, _zc, 0)

    ones_v[pl.ds(0, 16)] = jnp.ones((16,), jnp.float32)

    base = s * RPT
    for t in range(RPT // CB):
        pltpu.async_copy(m0, acc_sh.at[pl.ds(base + t * CB, CB)], sem_m0)
    for t in range(RPT // CB):
        pltpu.make_async_copy(m0, acc_sh.at[pl.ds(base, CB)], sem_m0).wait()
    pltpu.sync_copy(czero, cnt_sh.at[pl.ds(base, RPT)])
    plsc.subcore_barrier()

    def _wait(buf, sem):
        pltpu.make_async_copy(x_hbm.at[pl.ds(0, CB)], buf, sem).wait()

    def _drain_m(p):
        pltpu.make_async_copy(mbufs[p], acc_sh.at[idx_s.at[0]],
                              msems[p]).wait()
        pltpu.make_async_copy(ones_v.at[pl.ds(0, CB)], cnt_sh.at[idx_s.at[0]],
                              csems[p]).wait()

    def _compute(sb, db, mb):
        def _row(r, _):
            for j in range(D // 16):
                sl = pl.ds(j * 16, 16)
                mb[r, sl] = jnp.abs(sb[r, sl] - db[r, sl])
            return 0
        lax.fori_loop(0, CB, _row, 0)

    def _block(blk, _):
        pltpu.sync_copy(src_hbm.at[wid, blk], idx_s)
        pltpu.sync_copy(dst_hbm.at[wid, blk], idx_d)
        for i in range(DEPTH):
            pltpu.async_copy(x_hbm.at[idx_s.at[i]], sbufs[i], ssems[i])
            pltpu.async_copy(x_hbm.at[idx_d.at[i]], dbufs[i], dsems[i])

        def _round(kr, __):
            for i in range(DEPTH):
                t = DEPTH * kr + i
                p = i % 2
                _wait(sbufs[i], ssems[i])
                _wait(dbufs[i], dsems[i])
                if i < 2:
                    @pl.when(kr > 0)
                    def _():
                        _drain_m(p)
                else:
                    _drain_m(p)
                _compute(sbufs[i], dbufs[i], mbufs[p])
                pltpu.async_copy(mbufs[p], acc_sh.at[idx_s.at[t]],
                                 msems[p], add=True)
                pltpu.async_copy(ones_v.at[pl.ds(0, CB)],
                                 cnt_sh.at[idx_s.at[t]], csems[p], add=True)

                @pl.when(kr < BCH // DEPTH - 1)
                def _():
                    pltpu.async_copy(x_hbm.at[idx_s.at[t + DEPTH]],
                                     sbufs[i], ssems[i])
                    pltpu.async_copy(x_hbm.at[idx_d.at[t + DEPTH]],
                                     dbufs[i], dsems[i])
            return 0
        lax.fori_loop(0, BCH // DEPTH, _round, 0)
        _drain_m(0)
        _drain_m(1)
        return 0
    lax.fori_loop(0, NBLK, _block, 0)

    plsc.subcore_barrier()
    pltpu.sync_copy(acc_sh.at[pl.ds(base, RPT)],
                    acc_out.at[c].at[pl.ds(base, RPT)])
    pltpu.sync_copy(cnt_sh.at[pl.ds(base, RPT)],
                    cnt_out.at[pl.ds(c * NPAD + base, RPT)])


_sc_gather_scatter = pl.kernel(
    _sc_body,
    out_type=(
        jax.ShapeDtypeStruct((NC, NPAD, D), jnp.float32),
        jax.ShapeDtypeStruct((NC * NPAD,), jnp.float32),
    ),
    mesh=plsc.VectorSubcoreMesh(core_axis_name="c", subcore_axis_name="s"),
    scratch_types=(
        [pltpu.VMEM((BCH, CB), jnp.int32)] * 2
        + [pltpu.VMEM((CB, D), jnp.float32)] * (2 * DEPTH + 2)
        + [pltpu.VMEM((RPT,), jnp.float32), pltpu.VMEM((16,), jnp.float32)]
        + [pltpu.SemaphoreType.DMA] * (2 * DEPTH + 4)
        + [pltpu.VMEM_SHARED((NPAD, D), jnp.float32),
           pltpu.VMEM_SHARED((NPAD,), jnp.float32)]
    ),
)


BLK = 400


def _tc_body(x_ref, acc_ref, cnt_ref, w_ref, b_ref, o_ref):
    a = x_ref[...] + acc_ref[0] + acc_ref[1]
    denom = cnt_ref[...].sum(axis=1, keepdims=True) + 1.0
    m = a / denom
    o_ref[...] = lax.dot_general(
        m, w_ref[...], (((1,), (1,)), ((), ())),
        preferred_element_type=jnp.float32) + b_ref[...]


_tc_combine = pl.pallas_call(
    _tc_body,
    grid=(N // BLK,),
    in_specs=[
        pl.BlockSpec((BLK, D), lambda i: (i, 0)),
        pl.BlockSpec((NC, BLK, D), lambda i: (0, i, 0)),
        pl.BlockSpec((BLK, NC), lambda i: (i, 0)),
        pl.BlockSpec((D, D), lambda i: (0, 0)),
        pl.BlockSpec((1, D), lambda i: (0, 0)),
    ],
    out_specs=pl.BlockSpec((BLK, D), lambda i: (i, 0)),
    out_shape=jax.ShapeDtypeStruct((N, D), jnp.float32),
    compiler_params=pltpu.CompilerParams(
        dimension_semantics=("arbitrary",)),
)


def kernel(x, edge_index, W, b):
    src = edge_index[0].reshape(NW, NBLK, BCH, CB)
    dst = edge_index[1].reshape(NW, NBLK, BCH, CB)
    acc, cnt = _sc_gather_scatter(x, src, dst)
    return _tc_combine(x, acc, cnt.reshape(NC, NPAD).T, W, b.reshape(1, D))


# speedup vs baseline: 1.4259x; 1.4259x over previous
import jax
import jax.numpy as jnp
from jax import lax
from jax.experimental import pallas as pl
from jax.experimental.pallas import tpu as pltpu, tpu_sc as plsc

N = 10000
E = 320000
D = 128
NPAD = 10240
NC = 2
NS = 16
NW = NC * NS
EPW = E // NW
CB = 16
NCHUNK = EPW // CB
NBLK = 25
BCH = NCHUNK // NBLK
DEPTH = 5
RPT = NPAD // NS


def _sc_body(x_hbm, src_hbm, dst_hbm, acc_out, cnt_out, *refs):
    idx_s, idx_d = refs[0], refs[1]
    sbufs = refs[2:2 + DEPTH]
    dbufs = refs[2 + DEPTH:2 + 2 * DEPTH]
    m0, m1, czero, ones_v = refs[2 + 2 * DEPTH:6 + 2 * DEPTH]
    sems = refs[6 + 2 * DEPTH:]
    ssems = sems[0:DEPTH]
    dsems = sems[DEPTH:2 * DEPTH]
    sem_m0, sem_m1, sem_c0, sem_c1 = sems[2 * DEPTH:2 * DEPTH + 4]
    acc_sh, cnt_sh = sems[2 * DEPTH + 4:]
    mbufs = (m0, m1)
    msems = (sem_m0, sem_m1)
    csems = (sem_c0, sem_c1)

    c = lax.axis_index("c")
    s = lax.axis_index("s")
    wid = c * NS + s

    def _zrow(r, _):
        for j in range(D // 16):
            m0[r, pl.ds(j * 16, 16)] = jnp.zeros((16,), jnp.float32)
        return 0
    lax.fori_loop(0, CB, _zrow, 0)

    def _zc(r, _):
        czero[pl.ds(r * 16, 16)] = jnp.zeros((16,), jnp.float32)
        return 0
    lax.fori_loop(0, RPT // 16, _zc, 0)

    ones_v[pl.ds(0, 16)] = jnp.ones((16,), jnp.float32)

    base = s * RPT
    for t in range(RPT // CB):
        pltpu.async_copy(m0, acc_sh.at[pl.ds(base + t * CB, CB)], sem_m0)
    for t in range(RPT // CB):
        pltpu.make_async_copy(m0, acc_sh.at[pl.ds(base, CB)], sem_m0).wait()
    pltpu.sync_copy(czero, cnt_sh.at[pl.ds(base, RPT)])
    plsc.subcore_barrier()

    def _wait(buf, sem):
        pltpu.make_async_copy(x_hbm.at[pl.ds(0, CB)], buf, sem).wait()

    def _drain_m(p):
        pltpu.make_async_copy(mbufs[p], acc_sh.at[idx_s.at[0]],
                              msems[p]).wait()
        pltpu.make_async_copy(ones_v.at[pl.ds(0, CB)], cnt_sh.at[idx_s.at[0]],
                              csems[p]).wait()

    def _compute(sb, db, mb):
        def _row(r, _):
            for j in range(D // 16):
                sl = pl.ds(j * 16, 16)
                mb[r, sl] = jnp.abs(sb[r, sl] - db[r, sl])
            return 0
        lax.fori_loop(0, CB, _row, 0)

    def _block(blk, _):
        pltpu.sync_copy(src_hbm.at[wid, blk], idx_s)
        pltpu.sync_copy(dst_hbm.at[wid, blk], idx_d)
        for i in range(DEPTH):
            pltpu.async_copy(x_hbm.at[idx_s.at[i]], sbufs[i], ssems[i])
            pltpu.async_copy(x_hbm.at[idx_d.at[i]], dbufs[i], dsems[i])

        def _round(kr, __):
            for i in range(DEPTH):
                t = DEPTH * kr + i
                p = i % 2
                _wait(sbufs[i], ssems[i])
                _wait(dbufs[i], dsems[i])
                if i < 2:
                    @pl.when(kr > 0)
                    def _():
                        _drain_m(p)
                else:
                    _drain_m(p)
                _compute(sbufs[i], dbufs[i], mbufs[p])
                pltpu.async_copy(mbufs[p], acc_sh.at[idx_s.at[t]],
                                 msems[p], add=True)
                pltpu.async_copy(ones_v.at[pl.ds(0, CB)],
                                 cnt_sh.at[idx_s.at[t]], csems[p], add=True)

                @pl.when(kr < BCH // DEPTH - 1)
                def _():
                    pltpu.async_copy(x_hbm.at[idx_s.at[t + DEPTH]],
                                     sbufs[i], ssems[i])
                    pltpu.async_copy(x_hbm.at[idx_d.at[t + DEPTH]],
                                     dbufs[i], dsems[i])
            return 0
        lax.fori_loop(0, BCH // DEPTH, _round, 0)
        _drain_m(0)
        _drain_m(1)
        return 0
    lax.fori_loop(0, NBLK, _block, 0)

    plsc.subcore_barrier()
    pltpu.sync_copy(acc_sh.at[pl.ds(base, RPT)],
                    acc_out.at[c].at[pl.ds(base, RPT)])
    pltpu.sync_copy(cnt_sh.at[pl.ds(base, RPT)],
                    cnt_out.at[pl.ds(c * NPAD + base, RPT)])


_sc_gather_scatter = pl.kernel(
    _sc_body,
    out_type=(
        jax.ShapeDtypeStruct((NC, NPAD, D), jnp.float32),
        jax.ShapeDtypeStruct((NC * NPAD,), jnp.float32),
    ),
    mesh=plsc.VectorSubcoreMesh(core_axis_name="c", subcore_axis_name="s"),
    scratch_types=(
        [pltpu.VMEM((BCH, CB), jnp.int32)] * 2
        + [pltpu.VMEM((CB, D), jnp.float32)] * (2 * DEPTH + 2)
        + [pltpu.VMEM((RPT,), jnp.float32), pltpu.VMEM((16,), jnp.float32)]
        + [pltpu.SemaphoreType.DMA] * (2 * DEPTH + 4)
        + [pltpu.VMEM_SHARED((NPAD, D), jnp.float32),
           pltpu.VMEM_SHARED((NPAD,), jnp.float32)]
    ),
)


BLK = 400


def _tc_body(x_ref, acc_ref, cnt_ref, w_ref, b_ref, o_ref):
    a = x_ref[...] + acc_ref[0] + acc_ref[1]
    denom = cnt_ref[...].sum(axis=1, keepdims=True) + 1.0
    m = a / denom
    o_ref[...] = lax.dot_general(
        m, w_ref[...], (((1,), (1,)), ((), ())),
        preferred_element_type=jnp.float32) + b_ref[...]


_tc_combine = pl.pallas_call(
    _tc_body,
    grid=(N // BLK,),
    in_specs=[
        pl.BlockSpec((BLK, D), lambda i: (i, 0)),
        pl.BlockSpec((NC, BLK, D), lambda i: (0, i, 0)),
        pl.BlockSpec((BLK, NC), lambda i: (i, 0)),
        pl.BlockSpec((D, D), lambda i: (0, 0)),
        pl.BlockSpec((1, D), lambda i: (0, 0)),
    ],
    out_specs=pl.BlockSpec((BLK, D), lambda i: (i, 0)),
    out_shape=jax.ShapeDtypeStruct((N, D), jnp.float32),
    compiler_params=pltpu.CompilerParams(
        dimension_semantics=("arbitrary",)),
)


def kernel(x, edge_index, W, b):
    src = edge_index[0].reshape(NW, NBLK, BCH, CB)
    dst = edge_index[1].reshape(NW, NBLK, BCH, CB)
    acc, cnt = _sc_gather_scatter(x, src, dst)
    return _tc_combine(x, acc, cnt.reshape(NC, NPAD).T, W, b.reshape(1, D))


# speedup vs baseline: 1.4689x; 1.0301x over previous
import jax
import jax.numpy as jnp
from jax import lax
from jax.experimental import pallas as pl
from jax.experimental.pallas import tpu as pltpu, tpu_sc as plsc

N = 10000
E = 320000
D = 128
NPAD = 10240
NC = 2
NS = 16
NW = NC * NS
EPW = E // NW
CB = 16
NCHUNK = EPW // CB
NBLK = 25
BCH = NCHUNK // NBLK
DEPTH = 5
RPT = NPAD // NS


def _sc_body(x_hbm, src_hbm, dst_hbm, acc_out, cnt_out, *refs):
    idx_s, idx_d = refs[0], refs[1]
    sbufs = refs[2:2 + DEPTH]
    dbufs = refs[2 + DEPTH:2 + 2 * DEPTH]
    m0, m1, czero, ones_v = refs[2 + 2 * DEPTH:6 + 2 * DEPTH]
    sems = refs[6 + 2 * DEPTH:]
    ssems = sems[0:DEPTH]
    dsems = sems[DEPTH:2 * DEPTH]
    sem_m0, sem_m1, sem_c0, sem_c1 = sems[2 * DEPTH:2 * DEPTH + 4]
    acc_sh, cnt_sh = sems[2 * DEPTH + 4:]
    mbufs = (m0, m1)
    msems = (sem_m0, sem_m1)
    csems = (sem_c0, sem_c1)

    c = lax.axis_index("c")
    s = lax.axis_index("s")
    wid = c * NS + s

    def _zrow(r, _):
        for j in range(D // 16):
            m0[r, pl.ds(j * 16, 16)] = jnp.zeros((16,), jnp.float32)
        return 0
    lax.fori_loop(0, CB, _zrow, 0)

    def _zc(r, _):
        czero[pl.ds(r * 16, 16)] = jnp.zeros((16,), jnp.float32)
        return 0
    lax.fori_loop(0, RPT // 16, _zc, 0)

    ones_v[pl.ds(0, 16)] = jnp.ones((16,), jnp.float32)

    base = s * RPT
    for t in range(RPT // CB):
        pltpu.async_copy(m0, acc_sh.at[pl.ds(base + t * CB, CB)], sem_m0)
    for t in range(RPT // CB):
        pltpu.make_async_copy(m0, acc_sh.at[pl.ds(base, CB)], sem_m0).wait()
    pltpu.sync_copy(czero, cnt_sh.at[pl.ds(base, RPT)])
    plsc.subcore_barrier()

    def _wait(buf, sem):
        pltpu.make_async_copy(x_hbm.at[pl.ds(0, CB)], buf, sem).wait()

    def _drain_m(p):
        pltpu.make_async_copy(mbufs[p], acc_sh.at[idx_s.at[0]],
                              msems[p]).wait()
        pltpu.make_async_copy(ones_v.at[pl.ds(0, CB)], cnt_sh.at[idx_s.at[0]],
                              csems[p]).wait()

    def _compute(sb, db, mb):
        def _row(r, _):
            for j in range(D // 16):
                sl = pl.ds(j * 16, 16)
                mb[r, sl] = jnp.abs(sb[r, sl] - db[r, sl])
            return 0
        lax.fori_loop(0, CB, _row, 0)

    def _block(blk, _):
        pltpu.sync_copy(src_hbm.at[wid, blk], idx_s)
        pltpu.sync_copy(dst_hbm.at[wid, blk], idx_d)
        for i in range(DEPTH):
            pltpu.async_copy(x_hbm.at[idx_s.at[i]], sbufs[i], ssems[i])
            pltpu.async_copy(x_hbm.at[idx_d.at[i]], dbufs[i], dsems[i])

        def _round(kr, __):
            for i in range(DEPTH):
                t = DEPTH * kr + i
                p = i % 2
                _wait(sbufs[i], ssems[i])
                _wait(dbufs[i], dsems[i])
                if i < 2:
                    @pl.when(kr > 0)
                    def _():
                        _drain_m(p)
                else:
                    _drain_m(p)
                _compute(sbufs[i], dbufs[i], mbufs[p])
                pltpu.async_copy(mbufs[p], acc_sh.at[idx_s.at[t]],
                                 msems[p], add=True)
                pltpu.async_copy(ones_v.at[pl.ds(0, CB)],
                                 cnt_sh.at[idx_s.at[t]], csems[p], add=True)

                @pl.when(kr < BCH // DEPTH - 1)
                def _():
                    pltpu.async_copy(x_hbm.at[idx_s.at[t + DEPTH]],
                                     sbufs[i], ssems[i])
                    pltpu.async_copy(x_hbm.at[idx_d.at[t + DEPTH]],
                                     dbufs[i], dsems[i])
            return 0
        lax.fori_loop(0, BCH // DEPTH, _round, 0)
        _drain_m(0)
        _drain_m(1)
        return 0
    lax.fori_loop(0, NBLK, _block, 0)

    plsc.subcore_barrier()
    pltpu.sync_copy(acc_sh.at[pl.ds(base, RPT)],
                    acc_out.at[c].at[pl.ds(base, RPT)])
    pltpu.sync_copy(cnt_sh.at[pl.ds(base, RPT)],
                    cnt_out.at[pl.ds(c * NPAD + base, RPT)])


_sc_gather_scatter = pl.kernel(
    _sc_body,
    out_type=(
        jax.ShapeDtypeStruct((NC, NPAD, D), jnp.float32),
        jax.ShapeDtypeStruct((NC * NPAD,), jnp.float32),
    ),
    mesh=plsc.VectorSubcoreMesh(core_axis_name="c", subcore_axis_name="s"),
    scratch_types=(
        [pltpu.VMEM((BCH, CB), jnp.int32)] * 2
        + [pltpu.VMEM((CB, D), jnp.float32)] * (2 * DEPTH + 2)
        + [pltpu.VMEM((RPT,), jnp.float32), pltpu.VMEM((16,), jnp.float32)]
        + [pltpu.SemaphoreType.DMA] * (2 * DEPTH + 4)
        + [pltpu.VMEM_SHARED((NPAD, D), jnp.float32),
           pltpu.VMEM_SHARED((NPAD,), jnp.float32)]
    ),
)


BLK = 1000


def _tc_body(x_ref, acc_ref, cnt_ref, w_ref, b_ref, o_ref):
    a = x_ref[...] + acc_ref[0] + acc_ref[1]
    denom = cnt_ref[...].sum(axis=1, keepdims=True) + 1.0
    m = a / denom
    o_ref[...] = lax.dot_general(
        m, w_ref[...], (((1,), (1,)), ((), ())),
        preferred_element_type=jnp.float32) + b_ref[...]


_tc_combine = pl.pallas_call(
    _tc_body,
    grid=(N // BLK,),
    in_specs=[
        pl.BlockSpec((BLK, D), lambda i: (i, 0)),
        pl.BlockSpec((NC, BLK, D), lambda i: (0, i, 0)),
        pl.BlockSpec((BLK, NC), lambda i: (i, 0)),
        pl.BlockSpec((D, D), lambda i: (0, 0)),
        pl.BlockSpec((1, D), lambda i: (0, 0)),
    ],
    out_specs=pl.BlockSpec((BLK, D), lambda i: (i, 0)),
    out_shape=jax.ShapeDtypeStruct((N, D), jnp.float32),
    compiler_params=pltpu.CompilerParams(
        dimension_semantics=("arbitrary",)),
)


def kernel(x, edge_index, W, b):
    src = edge_index[0].reshape(NW, NBLK, BCH, CB)
    dst = edge_index[1].reshape(NW, NBLK, BCH, CB)
    acc, cnt = _sc_gather_scatter(x, src, dst)
    return _tc_combine(x, acc, cnt.reshape(NC, NPAD).T, W, b.reshape(1, D))


# speedup vs baseline: 1.4819x; 1.0089x over previous
import jax
import jax.numpy as jnp
from jax import lax
from jax.experimental import pallas as pl
from jax.experimental.pallas import tpu as pltpu, tpu_sc as plsc

N = 10000
E = 320000
D = 128
NPAD = 10240
NC = 2
NS = 16
NW = NC * NS
EPW = E // NW
CB = 16
NCHUNK = EPW // CB
NBLK = 25
BCH = NCHUNK // NBLK
DEPTH = 5
RPT = NPAD // NS


def _sc_body(x_hbm, src_hbm, dst_hbm, acc_out, cnt_out, *refs):
    idx_s, idx_d = refs[0], refs[1]
    sbufs = refs[2:2 + DEPTH]
    dbufs = refs[2 + DEPTH:2 + 2 * DEPTH]
    m0, m1, czero, ones_v = refs[2 + 2 * DEPTH:6 + 2 * DEPTH]
    sems = refs[6 + 2 * DEPTH:]
    ssems = sems[0:DEPTH]
    dsems = sems[DEPTH:2 * DEPTH]
    sem_m0, sem_m1, sem_c0, sem_c1 = sems[2 * DEPTH:2 * DEPTH + 4]
    acc_sh, cnt_sh = sems[2 * DEPTH + 4:]
    mbufs = (m0, m1)
    msems = (sem_m0, sem_m1)
    csems = (sem_c0, sem_c1)

    c = lax.axis_index("c")
    s = lax.axis_index("s")
    wid = c * NS + s

    def _zrow(r, _):
        for j in range(D // 16):
            m0[r, pl.ds(j * 16, 16)] = jnp.zeros((16,), jnp.float32)
        return 0
    lax.fori_loop(0, CB, _zrow, 0)

    def _zc(r, _):
        czero[pl.ds(r * 16, 16)] = jnp.zeros((16,), jnp.float32)
        return 0
    lax.fori_loop(0, RPT // 16, _zc, 0)

    ones_v[pl.ds(0, 16)] = jnp.ones((16,), jnp.float32)

    base = s * RPT
    for t in range(RPT // CB):
        pltpu.async_copy(m0, acc_sh.at[pl.ds(base + t * CB, CB)], sem_m0)
    for t in range(RPT // CB):
        pltpu.make_async_copy(m0, acc_sh.at[pl.ds(base, CB)], sem_m0).wait()
    pltpu.sync_copy(czero, cnt_sh.at[pl.ds(base, RPT)])
    plsc.subcore_barrier()

    def _wait(buf, sem):
        pltpu.make_async_copy(x_hbm.at[pl.ds(0, CB)], buf, sem).wait()

    def _drain_m(p):
        pltpu.make_async_copy(mbufs[p], acc_sh.at[idx_s.at[0]],
                              msems[p]).wait()
        pltpu.make_async_copy(ones_v.at[pl.ds(0, CB)], cnt_sh.at[idx_s.at[0]],
                              csems[p]).wait()

    def _compute(sb, db, mb):
        def _row(r, _):
            for j in range(D // 16):
                sl = pl.ds(j * 16, 16)
                mb[r, sl] = jnp.abs(sb[r, sl] - db[r, sl])
            return 0
        lax.fori_loop(0, CB, _row, 0)

    def _block(blk, _):
        pltpu.sync_copy(src_hbm.at[wid, blk], idx_s)
        pltpu.sync_copy(dst_hbm.at[wid, blk], idx_d)
        for i in range(DEPTH):
            pltpu.async_copy(x_hbm.at[idx_s.at[i]], sbufs[i], ssems[i])
            pltpu.async_copy(x_hbm.at[idx_d.at[i]], dbufs[i], dsems[i])

        def _round(kr, __):
            for i in range(DEPTH):
                t = DEPTH * kr + i
                p = i % 2
                _wait(sbufs[i], ssems[i])
                _wait(dbufs[i], dsems[i])
                if i < 2:
                    @pl.when(kr > 0)
                    def _():
                        _drain_m(p)
                else:
                    _drain_m(p)
                _compute(sbufs[i], dbufs[i], mbufs[p])
                pltpu.async_copy(mbufs[p], acc_sh.at[idx_s.at[t]],
                                 msems[p], add=True)
                pltpu.async_copy(ones_v.at[pl.ds(0, CB)],
                                 cnt_sh.at[idx_s.at[t]], csems[p], add=True)

                @pl.when(kr < BCH // DEPTH - 1)
                def _():
                    pltpu.async_copy(x_hbm.at[idx_s.at[t + DEPTH]],
                                     sbufs[i], ssems[i])
                    pltpu.async_copy(x_hbm.at[idx_d.at[t + DEPTH]],
                                     dbufs[i], dsems[i])
            return 0
        lax.fori_loop(0, BCH // DEPTH, _round, 0)
        _drain_m(0)
        _drain_m(1)
        return 0
    lax.fori_loop(0, NBLK, _block, 0)

    plsc.subcore_barrier()
    pltpu.sync_copy(acc_sh.at[pl.ds(base, RPT)],
                    acc_out.at[c].at[pl.ds(base, RPT)])
    pltpu.sync_copy(cnt_sh.at[pl.ds(base, RPT)],
                    cnt_out.at[pl.ds(c * NPAD + base, RPT)])


_sc_gather_scatter = pl.kernel(
    _sc_body,
    out_type=(
        jax.ShapeDtypeStruct((NC, NPAD, D), jnp.float32),
        jax.ShapeDtypeStruct((NC * NPAD,), jnp.float32),
    ),
    mesh=plsc.VectorSubcoreMesh(core_axis_name="c", subcore_axis_name="s"),
    scratch_types=(
        [pltpu.VMEM((BCH, CB), jnp.int32)] * 2
        + [pltpu.VMEM((CB, D), jnp.float32)] * (2 * DEPTH + 2)
        + [pltpu.VMEM((RPT,), jnp.float32), pltpu.VMEM((16,), jnp.float32)]
        + [pltpu.SemaphoreType.DMA] * (2 * DEPTH + 4)
        + [pltpu.VMEM_SHARED((NPAD, D), jnp.float32),
           pltpu.VMEM_SHARED((NPAD,), jnp.float32)]
    ),
)


BLK = 2000


def _tc_body(x_ref, acc_ref, cnt_ref, w_ref, b_ref, o_ref):
    a = x_ref[...] + acc_ref[0] + acc_ref[1]
    denom = cnt_ref[...].sum(axis=1, keepdims=True) + 1.0
    m = a / denom
    o_ref[...] = lax.dot_general(
        m, w_ref[...], (((1,), (1,)), ((), ())),
        preferred_element_type=jnp.float32) + b_ref[...]


_tc_combine = pl.pallas_call(
    _tc_body,
    grid=(N // BLK,),
    in_specs=[
        pl.BlockSpec((BLK, D), lambda i: (i, 0)),
        pl.BlockSpec((NC, BLK, D), lambda i: (0, i, 0)),
        pl.BlockSpec((BLK, NC), lambda i: (i, 0)),
        pl.BlockSpec((D, D), lambda i: (0, 0)),
        pl.BlockSpec((1, D), lambda i: (0, 0)),
    ],
    out_specs=pl.BlockSpec((BLK, D), lambda i: (i, 0)),
    out_shape=jax.ShapeDtypeStruct((N, D), jnp.float32),
    compiler_params=pltpu.CompilerParams(
        dimension_semantics=("arbitrary",)),
)


def kernel(x, edge_index, W, b):
    src = edge_index[0].reshape(NW, NBLK, BCH, CB)
    dst = edge_index[1].reshape(NW, NBLK, BCH, CB)
    acc, cnt = _sc_gather_scatter(x, src, dst)
    return _tc_combine(x, acc, cnt.reshape(NC, NPAD).T, W, b.reshape(1, D))
